# baseline (speedup 1.0000x reference)
import functools
import jax
import jax.numpy as jnp
from jax import lax
from jax.experimental import pallas as pl
from jax.experimental.pallas import tpu as pltpu
from jax.experimental.pallas import tpu_sc as plsc

_NC, _NS = 2, 16
_NW = _NC * _NS


def _sc_copy(probs_hbm, out_hbm, buf_v):
    wid = lax.axis_index("s") * _NC + lax.axis_index("c")
    base = wid * 8
    pltpu.sync_copy(probs_hbm.at[pl.ds(base, 8)], buf_v)
    pltpu.sync_copy(buf_v, out_hbm.at[pl.ds(base, 8)])


def kernel(x, top_k_probs, top_k_indices, router_logits, w_gate, w_noise):
    t, k = top_k_probs.shape
    mesh = plsc.VectorSubcoreMesh(core_axis_name="c", subcore_axis_name="s")
    f = functools.partial(
        pl.kernel,
        mesh=mesh,
        out_type=jax.ShapeDtypeStruct((t, k), top_k_probs.dtype),
        scratch_types=[pltpu.VMEM((8, k), top_k_probs.dtype)],
    )(_sc_copy)
    return f(top_k_probs)


# speedup vs baseline: 1.1861x; 1.1861x over previous
import jax
import jax.numpy as jnp
from jax.experimental import pallas as pl
from jax.experimental.pallas import tpu as pltpu

_CHUNKS = 16


def _copy_kernel(probs_hbm, out_hbm, buf_v, in_sems, out_sems):
    rows = probs_hbm.shape[0]
    chunk = rows // _CHUNKS
    ins = []
    for c in range(_CHUNKS):
        cp = pltpu.make_async_copy(
            probs_hbm.at[pl.ds(c * chunk, chunk)],
            buf_v.at[pl.ds(c * chunk, chunk)],
            in_sems.at[c],
        )
        cp.start()
        ins.append(cp)
    outs = []
    for c in range(_CHUNKS):
        ins[c].wait()
        cp = pltpu.make_async_copy(
            buf_v.at[pl.ds(c * chunk, chunk)],
            out_hbm.at[pl.ds(c * chunk, chunk)],
            out_sems.at[c],
        )
        cp.start()
        outs.append(cp)
    for c in range(_CHUNKS):
        outs[c].wait()


def kernel(x, top_k_probs, top_k_indices, router_logits, w_gate, w_noise):
    t, k = top_k_probs.shape
    return pl.pallas_call(
        _copy_kernel,
        in_specs=[pl.BlockSpec(memory_space=pltpu.MemorySpace.HBM)],
        out_specs=pl.BlockSpec(memory_space=pltpu.MemorySpace.HBM),
        scratch_shapes=[
            pltpu.VMEM((t, k), top_k_probs.dtype),
            pltpu.SemaphoreType.DMA((_CHUNKS,)),
            pltpu.SemaphoreType.DMA((_CHUNKS,)),
        ],
        out_shape=jax.ShapeDtypeStruct((t, k), top_k_probs.dtype),
    )(top_k_probs)
